# baseline (speedup 1.0000x reference)
import jax
import jax.numpy as jnp
from jax.experimental import pallas as pl
from jax.experimental.pallas import tpu as pltpu

B, H, W = 8, 128, 128
C, K, D = 19, 1, 512
Q = B * H * W
BQ = 2048
BH = BQ // W
PB = H // BH


def _fused_body(emb_ref, logits_ref, means_ref, npw_ref,
                py_ref, ps_ref, score_ref, nm_ref, lgout_ref):
    e = emb_ref[...]
    m = means_ref[:, 0, :]
    lg = logits_ref[...]
    w = npw_ref[...]
    lgT = lg.T

    crossT = jax.lax.dot_general(
        m, e, (((1,), (1,)), ((), ())),
        preferred_element_type=jnp.float32)
    ee = e * e
    q2T = jax.lax.dot_general(
        jnp.ones((1, D), jnp.float32), ee, (((1,), (1,)), ((), ())),
        preferred_element_type=jnp.float32)
    m2 = jnp.sum(m * m, axis=1, keepdims=True)

    nmT = q2T + m2 - 2.0 * crossT
    simT = 1.0 / (1.0 + 0.5 * nmT)
    w0 = w[:, 0:1]
    w1 = w[:, 1:2]
    w2 = w[:, 2:3]
    scoreT = jax.nn.sigmoid(w0 * lgT + w1 * simT + w2)

    mxT = jnp.max(lgT, axis=0, keepdims=True)
    iotaT = jax.lax.broadcasted_iota(jnp.int32, lgT.shape, 0)
    pyT = jnp.min(jnp.where(lgT == mxT, iotaT, C), axis=0,
                  keepdims=True)
    onehotT = iotaT == pyT
    psT = jnp.sum(jnp.where(onehotT, scoreT, 0.0), axis=0,
                  keepdims=True)

    py_ref[...] = pyT.astype(jnp.float32).reshape(1, BH, W)
    ps_ref[...] = psT.reshape(1, BH, W)
    score_ref[...] = scoreT.T.reshape(1, BH, W, C)
    nm_ref[...] = nmT.T.reshape(1, BH, W, C)
    lgout_ref[...] = lg.reshape(1, BH, W, C)


def kernel(emb, logits, means, np_w):
    grid = (Q // BQ,)
    py, ps, score, nm, lgout = pl.pallas_call(
        _fused_body,
        grid=grid,
        in_specs=[
            pl.BlockSpec((BQ, D), lambda i: (i, 0)),
            pl.BlockSpec((BQ, C), lambda i: (i, 0)),
            pl.BlockSpec((C, K, D), lambda i: (0, 0, 0)),
            pl.BlockSpec((C, 3), lambda i: (0, 0)),
        ],
        out_specs=[
            pl.BlockSpec((1, BH, W), lambda i: (i // PB, i % PB, 0)),
            pl.BlockSpec((1, BH, W), lambda i: (i // PB, i % PB, 0)),
            pl.BlockSpec((1, BH, W, C), lambda i: (i // PB, i % PB, 0, 0)),
            pl.BlockSpec((1, BH, W, C), lambda i: (i // PB, i % PB, 0, 0)),
            pl.BlockSpec((1, BH, W, C), lambda i: (i // PB, i % PB, 0, 0)),
        ],
        out_shape=[
            jax.ShapeDtypeStruct((B, H, W), jnp.float32),
            jax.ShapeDtypeStruct((B, H, W), jnp.float32),
            jax.ShapeDtypeStruct((B, H, W, C), jnp.float32),
            jax.ShapeDtypeStruct((B, H, W, C), jnp.float32),
            jax.ShapeDtypeStruct((B, H, W, C), jnp.float32),
        ],
        compiler_params=pltpu.CompilerParams(
            dimension_semantics=("parallel",)),
    )(emb, logits, means, np_w)

    return (py, ps, score, nm, lgout)


# speedup vs baseline: 1.6196x; 1.6196x over previous
import jax
import jax.numpy as jnp
from jax.experimental import pallas as pl
from jax.experimental.pallas import tpu as pltpu

B, H, W = 8, 128, 128
C, K, D = 19, 1, 512
Q = B * H * W
BQ = 4096
BH = BQ // W
PB = H // BH


def _fused_body(emb_ref, logits_ref, means_ref, npw_ref,
                nm_ref, score_ref, py_ref, ps_ref):
    e = emb_ref[...]
    m = means_ref[:, 0, :]
    lgT = logits_ref[...]
    w = npw_ref[...]

    crossT = jax.lax.dot_general(
        m, e, (((1,), (1,)), ((), ())),
        preferred_element_type=jnp.float32)
    ee = e * e
    q2T = jax.lax.dot_general(
        jnp.ones((1, D), jnp.float32), ee, (((1,), (1,)), ((), ())),
        preferred_element_type=jnp.float32)
    m2 = jnp.sum(m * m, axis=1, keepdims=True)

    nmT = q2T + m2 - 2.0 * crossT
    simT = 1.0 / (1.0 + 0.5 * nmT)
    w0 = w[:, 0:1]
    w1 = w[:, 1:2]
    w2 = w[:, 2:3]
    scoreT = jax.nn.sigmoid(w0 * lgT + w1 * simT + w2)

    mxT = jnp.max(lgT, axis=0, keepdims=True)
    iotaT = jax.lax.broadcasted_iota(jnp.int32, lgT.shape, 0)
    pyT = jnp.min(jnp.where(lgT == mxT, iotaT, C), axis=0,
                  keepdims=True)
    onehotT = iotaT == pyT
    psT = jnp.sum(jnp.where(onehotT, scoreT, 0.0), axis=0,
                  keepdims=True)

    nm_ref[...] = nmT
    score_ref[...] = scoreT
    py_ref[...] = pyT.astype(jnp.float32).reshape(1, BH, W)
    ps_ref[...] = psT.reshape(1, BH, W)


def kernel(emb, logits, means, np_w):
    logitsT = logits.T
    grid = (Q // BQ,)
    nmT, scoreT, py, ps = pl.pallas_call(
        _fused_body,
        grid=grid,
        in_specs=[
            pl.BlockSpec((BQ, D), lambda i: (i, 0)),
            pl.BlockSpec((C, BQ), lambda i: (0, i)),
            pl.BlockSpec((C, K, D), lambda i: (0, 0, 0)),
            pl.BlockSpec((C, 3), lambda i: (0, 0)),
        ],
        out_specs=[
            pl.BlockSpec((C, BQ), lambda i: (0, i)),
            pl.BlockSpec((C, BQ), lambda i: (0, i)),
            pl.BlockSpec((1, BH, W), lambda i: (i // PB, i % PB, 0)),
            pl.BlockSpec((1, BH, W), lambda i: (i // PB, i % PB, 0)),
        ],
        out_shape=[
            jax.ShapeDtypeStruct((C, Q), jnp.float32),
            jax.ShapeDtypeStruct((C, Q), jnp.float32),
            jax.ShapeDtypeStruct((B, H, W), jnp.float32),
            jax.ShapeDtypeStruct((B, H, W), jnp.float32),
        ],
        compiler_params=pltpu.CompilerParams(
            dimension_semantics=("parallel",)),
    )(emb, logitsT, means, np_w)

    pred_score_all = scoreT.T.reshape(B, H, W, C)
    nm_dist_r = nmT.T.reshape(B, H, W, C)
    logits_r = logits.reshape(B, H, W, C)
    return (py, ps, pred_score_all, nm_dist_r, logits_r)


# speedup vs baseline: 1.6901x; 1.0435x over previous
import jax
import jax.numpy as jnp
from jax.experimental import pallas as pl
from jax.experimental.pallas import tpu as pltpu

B, H, W = 8, 128, 128
C, K, D = 19, 1, 512
Q = B * H * W
BQ = 8192
BH = BQ // W
PB = H // BH


def _fused_body(emb_ref, logits_ref, means_ref, npw_ref,
                nm_ref, score_ref, py_ref, ps_ref):
    e = emb_ref[...]
    m = means_ref[:, 0, :]
    lgT = logits_ref[...]
    w = npw_ref[...]

    crossT = jax.lax.dot_general(
        m, e, (((1,), (1,)), ((), ())),
        preferred_element_type=jnp.float32)
    ee = e * e
    q2T = jax.lax.dot_general(
        jnp.ones((1, D), jnp.float32), ee, (((1,), (1,)), ((), ())),
        preferred_element_type=jnp.float32)
    m2 = jnp.sum(m * m, axis=1, keepdims=True)

    nmT = q2T + m2 - 2.0 * crossT
    simT = 1.0 / (1.0 + 0.5 * nmT)
    w0 = w[:, 0:1]
    w1 = w[:, 1:2]
    w2 = w[:, 2:3]
    scoreT = jax.nn.sigmoid(w0 * lgT + w1 * simT + w2)

    mxT = jnp.max(lgT, axis=0, keepdims=True)
    iotaT = jax.lax.broadcasted_iota(jnp.int32, lgT.shape, 0)
    pyT = jnp.min(jnp.where(lgT == mxT, iotaT, C), axis=0,
                  keepdims=True)
    onehotT = iotaT == pyT
    psT = jnp.sum(jnp.where(onehotT, scoreT, 0.0), axis=0,
                  keepdims=True)

    nm_ref[...] = nmT
    score_ref[...] = scoreT
    py_ref[...] = pyT.astype(jnp.float32).reshape(1, BH, W)
    ps_ref[...] = psT.reshape(1, BH, W)


def kernel(emb, logits, means, np_w):
    logitsT = logits.T
    grid = (Q // BQ,)
    nmT, scoreT, py, ps = pl.pallas_call(
        _fused_body,
        grid=grid,
        in_specs=[
            pl.BlockSpec((BQ, D), lambda i: (i, 0)),
            pl.BlockSpec((C, BQ), lambda i: (0, i)),
            pl.BlockSpec((C, K, D), lambda i: (0, 0, 0)),
            pl.BlockSpec((C, 3), lambda i: (0, 0)),
        ],
        out_specs=[
            pl.BlockSpec((C, BQ), lambda i: (0, i)),
            pl.BlockSpec((C, BQ), lambda i: (0, i)),
            pl.BlockSpec((1, BH, W), lambda i: (i // PB, i % PB, 0)),
            pl.BlockSpec((1, BH, W), lambda i: (i // PB, i % PB, 0)),
        ],
        out_shape=[
            jax.ShapeDtypeStruct((C, Q), jnp.float32),
            jax.ShapeDtypeStruct((C, Q), jnp.float32),
            jax.ShapeDtypeStruct((B, H, W), jnp.float32),
            jax.ShapeDtypeStruct((B, H, W), jnp.float32),
        ],
        compiler_params=pltpu.CompilerParams(
            dimension_semantics=("parallel",)),
    )(emb, logitsT, means, np_w)

    pred_score_all = scoreT.T.reshape(B, H, W, C)
    nm_dist_r = nmT.T.reshape(B, H, W, C)
    logits_r = logits.reshape(B, H, W, C)
    return (py, ps, pred_score_all, nm_dist_r, logits_r)
